# baseline (speedup 1.0000x reference)
import functools

import jax
import jax.numpy as jnp
from jax import lax
from jax.experimental import pallas as pl
from jax.experimental.pallas import tpu as pltpu
from jax.experimental.pallas import tpu_sc as plsc

B = 4096
L = 200
D = 64
H = 50
S = 4
NW = 32
BPW = B // NW
NBUF = 4
NC = 2

_mesh = plsc.VectorSubcoreMesh(core_axis_name="c", subcore_axis_name="s")


@functools.partial(
    pl.kernel,
    mesh=_mesh,
    out_type=jax.ShapeDtypeStruct((B, D), jnp.float32),
    compiler_params=pltpu.CompilerParams(use_tc_tiling_on_sc=False),
    scratch_types=[
        pltpu.VMEM((S * BPW, H), jnp.int32),
        pltpu.VMEM((BPW, 16), jnp.float32),
        pltpu.VMEM((NBUF, S * H, D), jnp.float32),
        pltpu.VMEM((BPW, D), jnp.float32),
        pltpu.SemaphoreType.DMA,
        pltpu.SemaphoreType.DMA,
        pltpu.SemaphoreType.DMA,
        pltpu.SemaphoreType.DMA,
    ],
)
def _bow_sc(table, idx_hbm, recip_hbm, out_hbm, idx_v, recip_v, bufs, out_v,
            sem0, sem1, sem2, sem3):
    sems = (sem0, sem1, sem2, sem3)
    wid = lax.axis_index("s") * NC + lax.axis_index("c")
    pltpu.sync_copy(idx_hbm.at[pl.ds(wid * (S * BPW), S * BPW)], idx_v)
    pltpu.sync_copy(recip_hbm.at[pl.ds(wid * BPW, BPW)], recip_v)

    def fire(bag, k):
        for j in range(S):
            pltpu.make_async_copy(
                table.at[idx_v.at[S * bag + j]],
                bufs.at[k, pl.ds(j * H, H)], sems[k]).start()

    def wait(k):
        for j in range(S):
            pltpu.make_async_copy(
                table.at[idx_v.at[0]], bufs.at[k, pl.ds(j * H, H)],
                sems[k]).wait()

    def reduce_bag(bag, k):
        def body(r, acc):
            a0, a1, a2, a3 = acc
            a0 = a0 + bufs[k, r, pl.ds(0, 16)]
            a1 = a1 + bufs[k, r, pl.ds(16, 16)]
            a2 = a2 + bufs[k, r, pl.ds(32, 16)]
            a3 = a3 + bufs[k, r, pl.ds(48, 16)]
            return a0, a1, a2, a3

        z = jnp.zeros((16,), jnp.float32)
        a0, a1, a2, a3 = lax.fori_loop(0, S * H, body, (z, z, z, z),
                                       unroll=25)
        rc = recip_v[bag, pl.ds(0, 16)]
        out_v[bag, pl.ds(0, 16)] = a0 * rc
        out_v[bag, pl.ds(16, 16)] = a1 * rc
        out_v[bag, pl.ds(32, 16)] = a2 * rc
        out_v[bag, pl.ds(48, 16)] = a3 * rc

    for k in range(NBUF - 1):
        fire(jnp.int32(k), k)

    def outer(g, carry):
        base = g * NBUF
        for k in range(NBUF):
            bag = base + k
            wait(k)
            nxt = bag + (NBUF - 1)

            @pl.when(nxt < BPW)
            def _():
                fire(nxt, (k + NBUF - 1) % NBUF)

            reduce_bag(bag, k)
        return carry

    lax.fori_loop(0, BPW // NBUF, outer, 0)
    pltpu.sync_copy(out_v, out_hbm.at[pl.ds(wid * BPW, BPW)])


def kernel(x, length, emb_weight):
    idx = x.astype(jnp.int32).reshape(S * B, H)
    recip = jnp.broadcast_to((1.0 / length.astype(jnp.float32))[:, None],
                             (B, 16))
    return _bow_sc(emb_weight, idx, recip)
